# Initial kernel scaffold; baseline (speedup 1.0000x reference)
import jax
import jax.numpy as jnp
from jax.experimental import pallas as pl


def kernel(atoms, connections, batch, We, be, W0, b0, W1, b1, W2, b2, W3, b3, W4, b4):
    raise NotImplementedError("write your pallas kernel here")


# speedup vs baseline: 4.6942x; 4.6942x over previous
import functools

import jax
import jax.numpy as jnp
from jax import lax
from jax.experimental import pallas as pl
from jax.experimental.pallas import tpu as pltpu
from jax.experimental.pallas import tpu_sc as plsc

N = 10000
E = 320000
B = 100
M = 200
D = 128

NC = 2
NS = 16
NW = NC * NS

NP = 10240
EC = 128
EPAD = 327680
EPW = EPAD // NW
ECH = EPW // EC

NZS = NP // NS

OFF = 10100
SPAD = 331776
SPW = SPAD // NW
SCH = SPW // EC

GP = 20480
GPW = GP // NW
GC = 128
GCH = GPW // GC

BM = 1024


def _mesh():
    return plsc.VectorSubcoreMesh(core_axis_name="c", subcore_axis_name="s",
                                  num_cores=NC, num_subcores=NS)


def _sc_stats(sidx_all, ones_hbm, zeros_hbm):
    @functools.partial(
        pl.kernel,
        out_type=jax.ShapeDtypeStruct((NC, NP, D), jnp.float32),
        mesh=_mesh(),
        scratch_types=[
            pltpu.VMEM((EC,), jnp.int32),
            pltpu.VMEM((EC, D), jnp.float32),
            pltpu.VMEM_SHARED((NP, D), jnp.float32),
        ],
    )
    def k(idx_h, ones_h, zeros_h, out_h, didx, rows, acc):
        c = lax.axis_index("c")
        s = lax.axis_index("s")
        w = s * NC + c
        pltpu.sync_copy(ones_h, rows)
        pltpu.sync_copy(zeros_h, acc.at[pl.ds(s * NZS, NZS)])
        plsc.subcore_barrier()

        base = w * SPW

        def body(i, carry):
            pltpu.sync_copy(idx_h.at[pl.ds(base + i * EC, EC)], didx)
            pltpu.sync_copy(rows, acc.at[didx], add=True)
            return carry

        lax.fori_loop(0, SCH, body, 0)

        plsc.subcore_barrier()
        pltpu.sync_copy(acc.at[pl.ds(s * NZS, NZS)],
                        out_h.at[c, pl.ds(s * NZS, NZS)])

    return k(sidx_all, ones_hbm, zeros_hbm)


def _sc_msgpass(z, src_p, dst_p, zeros_hbm):
    @functools.partial(
        pl.kernel,
        out_type=jax.ShapeDtypeStruct((NC, NP, D), jnp.float32),
        mesh=_mesh(),
        scratch_types=[
            pltpu.VMEM((EC,), jnp.int32),
            pltpu.VMEM((EC,), jnp.int32),
            pltpu.VMEM((EC, D), jnp.float32),
            pltpu.VMEM_SHARED((NP, D), jnp.float32),
            pltpu.SemaphoreType.DMA,
        ],
    )
    def k(z_h, src_h, dst_h, zeros_h, out_h, sidx, didx, rows, acc, sem):
        c = lax.axis_index("c")
        s = lax.axis_index("s")
        w = s * NC + c
        pltpu.sync_copy(zeros_h, acc.at[pl.ds(s * NZS, NZS)])
        plsc.subcore_barrier()

        ebase = w * EPW

        def body(i, carry):
            off = ebase + i * EC
            pltpu.sync_copy(src_h.at[pl.ds(off, EC)], sidx)
            pltpu.sync_copy(dst_h.at[pl.ds(off, EC)], didx)
            pltpu.async_copy(z_h.at[sidx], rows, sem).wait()
            pltpu.sync_copy(rows, acc.at[didx], add=True)
            return carry

        lax.fori_loop(0, ECH, body, 0)

        plsc.subcore_barrier()
        pltpu.sync_copy(acc.at[pl.ds(s * NZS, NZS)],
                        out_h.at[c, pl.ds(s * NZS, NZS)])

    return k(z, src_p, dst_p, zeros_hbm)


def _sc_dense(xpad, gidx):
    @functools.partial(
        pl.kernel,
        out_type=jax.ShapeDtypeStruct((GP, D), jnp.float32),
        mesh=_mesh(),
        scratch_types=[
            pltpu.VMEM((GC,), jnp.int32),
            pltpu.VMEM((GC, D), jnp.float32),
            pltpu.SemaphoreType.DMA,
        ],
    )
    def k(x_h, gidx_h, out_h, idx_v, rows, sem):
        c = lax.axis_index("c")
        s = lax.axis_index("s")
        w = s * NC + c
        base = w * GPW

        def body(i, carry):
            off = base + i * GC
            pltpu.sync_copy(gidx_h.at[pl.ds(off, GC)], idx_v)
            pltpu.async_copy(x_h.at[idx_v], rows, sem).wait()
            pltpu.sync_copy(rows, out_h.at[pl.ds(off, GC)])
            return carry

        lax.fori_loop(0, GCH, body, 0)

    return k(xpad, gidx)


def _tc_prep(atoms_p, Wp, be2, degp):
    def body(a_ref, w_ref, b_ref, d_ref, z_ref, dinv_ref):
        x = jnp.log(a_ref[...] + 1.0)
        x = jnp.dot(x, w_ref[...], preferred_element_type=jnp.float32)
        x = x + b_ref[...]
        deg = d_ref[0, :, 0:1] + d_ref[1, :, 0:1] + 2.0
        dinv = lax.rsqrt(deg)
        dinvf = jnp.broadcast_to(dinv, x.shape)
        z_ref[...] = dinvf * x
        dinv_ref[...] = dinvf

    return pl.pallas_call(
        body,
        grid=(NP // BM,),
        in_specs=[
            pl.BlockSpec((BM, D), lambda i: (i, 0)),
            pl.BlockSpec((D, D), lambda i: (0, 0)),
            pl.BlockSpec((1, D), lambda i: (0, 0)),
            pl.BlockSpec((NC, BM, D), lambda i: (0, i, 0)),
        ],
        out_specs=[pl.BlockSpec((BM, D), lambda i: (i, 0))] * 2,
        out_shape=[jax.ShapeDtypeStruct((NP, D), jnp.float32)] * 2,
    )(atoms_p, Wp, be2, degp)


def _tc_combine(p, z, dinvf, W, b2):
    def body(p_ref, z_ref, di_ref, w_ref, b_ref, x_ref, zn_ref):
        di = di_ref[...]
        y = di * (p_ref[0] + p_ref[1] + 2.0 * z_ref[...])
        h = jnp.dot(y, w_ref[...], preferred_element_type=jnp.float32)
        x = jnp.maximum(h + b_ref[...], 0.0)
        x_ref[...] = x
        zn_ref[...] = di * x

    return pl.pallas_call(
        body,
        grid=(NP // BM,),
        in_specs=[
            pl.BlockSpec((NC, BM, D), lambda i: (0, i, 0)),
            pl.BlockSpec((BM, D), lambda i: (i, 0)),
            pl.BlockSpec((BM, D), lambda i: (i, 0)),
            pl.BlockSpec((D, D), lambda i: (0, 0)),
            pl.BlockSpec((1, D), lambda i: (0, 0)),
        ],
        out_specs=[pl.BlockSpec((BM, D), lambda i: (i, 0))] * 2,
        out_shape=[jax.ShapeDtypeStruct((NP, D), jnp.float32)] * 2,
    )(p, z, dinvf, W, b2)


def kernel(atoms, connections, batch, We, be,
           W0, b0, W1, b1, W2, b2, W3, b3, W4, b4):
    src = connections[0]
    dst = connections[1]
    npad = EPAD - E
    src_p = jnp.concatenate([src, jnp.zeros((npad,), jnp.int32)])
    dst_p = jnp.concatenate([dst, jnp.full((npad,), N, jnp.int32)])
    sidx_all = jnp.concatenate(
        [dst, batch + OFF, jnp.full((SPAD - E - N,), N, jnp.int32)])

    atoms_p = jnp.pad(atoms, ((0, NP - N), (0, D - atoms.shape[1])))
    Wp = jnp.pad(We, ((0, D - We.shape[0]), (0, 0)))
    zerosD = jnp.zeros((NZS, D), jnp.float32)
    onesD = jnp.ones((EC, D), jnp.float32)

    statp = _sc_stats(sidx_all, onesD, zerosD)
    z, dinvf = _tc_prep(atoms_p, Wp, be.reshape(1, D), statp)

    x = z
    for Wl, bl in ((W0, b0), (W1, b1), (W2, b2), (W3, b3), (W4, b4)):
        p = _sc_msgpass(z, src_p, dst_p, zerosD)
        x, z = _tc_combine(p, z, dinvf, Wl, bl.reshape(1, D))

    counts = (statp[0, OFF:OFF + B, 0]
              + statp[1, OFF:OFF + B, 0]).astype(jnp.int32)
    starts = jnp.concatenate(
        [jnp.zeros((1,), jnp.int32), jnp.cumsum(counts)[:-1]])
    mcol = jnp.arange(M, dtype=jnp.int32)[None, :]
    mask = mcol < counts[:, None]
    gidx = jnp.where(mask, starts[:, None] + mcol, N).reshape(-1)
    gidx = jnp.concatenate([gidx, jnp.full((GP - B * M,), N, jnp.int32)])

    xpad = jnp.concatenate([x[:N], jnp.zeros((16, D), x.dtype)], axis=0)
    dense_flat = _sc_dense(xpad, gidx)
    dense = dense_flat[:B * M].reshape(B, M, D)
    return dense, mask
